# baseline (speedup 1.0000x reference)
import functools

import jax
import jax.numpy as jnp
from jax import lax
from jax.experimental import pallas as pl
from jax.experimental.pallas import tpu as pltpu
from jax.experimental.pallas import tpu_sc as plsc

_NC = 2
_NS = 16


def _embed_pool_sc(context_idxs, emb_table):
    B, CTX = context_idxs.shape
    _, EMB = emb_table.shape
    NW = _NC * _NS
    per_w = (B * CTX) // NW
    CH = 80
    n_chunks = per_w // CH
    b_per_w = B // NW

    idx_flat = context_idxs.astype(jnp.int32).reshape(NW, per_w)
    mesh = plsc.VectorSubcoreMesh(core_axis_name="c", subcore_axis_name="s")

    @functools.partial(
        pl.kernel,
        mesh=mesh,
        out_type=jax.ShapeDtypeStruct((B, EMB), jnp.float32),
        scratch_types=[
            pltpu.VMEM((per_w,), jnp.int32),
            pltpu.VMEM((per_w, EMB), jnp.float32),
            pltpu.VMEM((b_per_w, EMB), jnp.float32),
        ],
        compiler_params=pltpu.CompilerParams(use_tc_tiling_on_sc=False),
    )
    def gather_pool(table_hbm, idx_hbm, out_hbm, idx_v, rows_v, emb_v):
        wid = lax.axis_index("s") * _NC + lax.axis_index("c")
        pltpu.sync_copy(idx_hbm.at[wid], idx_v)

        @pl.loop(0, n_chunks)
        def _(c):
            sl = pl.ds(c * CH, CH)
            pltpu.sync_copy(table_hbm.at[idx_v.at[sl]], rows_v.at[sl])

        @pl.loop(0, b_per_w)
        def _(j):
            base = j * CTX
            acc = rows_v[base]
            for c in range(1, CTX):
                acc = acc + rows_v[base + c]
            emb_v[j] = acc

        pltpu.sync_copy(emb_v, out_hbm.at[pl.ds(wid * b_per_w, b_per_w)])

    return gather_pool(emb_table, idx_flat)


def _mm_body(emb_ref, w_ref, b_ref, out_ref):
    out_ref[...] = lax.dot_general(
        emb_ref[...], w_ref[...],
        dimension_numbers=(((1,), (1,)), ((), ())),
        preferred_element_type=jnp.float32,
    ) + b_ref[...]


def _project_tc(embedded, W, b):
    B, EMB = embedded.shape
    V = W.shape[0]
    TV = 2048
    grid = pl.cdiv(V, TV)
    return pl.pallas_call(
        _mm_body,
        grid=(grid,),
        in_specs=[
            pl.BlockSpec((B, EMB), lambda i: (0, 0)),
            pl.BlockSpec((TV, EMB), lambda i: (i, 0)),
            pl.BlockSpec((1, TV), lambda i: (0, i)),
        ],
        out_specs=pl.BlockSpec((B, TV), lambda i: (0, i)),
        out_shape=jax.ShapeDtypeStruct((B, V), jnp.float32),
        compiler_params=pltpu.CompilerParams(
            dimension_semantics=("parallel",),
        ),
    )(embedded, W, b.reshape(1, V))


def kernel(context_idxs, emb_table, W, b):
    embedded = _embed_pool_sc(context_idxs, emb_table)
    return _project_tc(embedded, W, b)


# speedup vs baseline: 2.7865x; 2.7865x over previous
import functools

import jax
import jax.numpy as jnp
from jax import lax
from jax.experimental import pallas as pl
from jax.experimental.pallas import tpu as pltpu
from jax.experimental.pallas import tpu_sc as plsc

_NC = 2
_NS = 16


def _embed_pool_sc(context_idxs, emb_table):
    B, CTX = context_idxs.shape
    _, EMB = emb_table.shape
    NW = _NC * _NS
    per_w = (B * CTX) // NW
    CH = 80
    n_chunks = per_w // CH
    b_per_w = B // NW

    idx_flat = context_idxs.astype(jnp.int32).reshape(NW, per_w)
    mesh = plsc.VectorSubcoreMesh(core_axis_name="c", subcore_axis_name="s")

    @functools.partial(
        pl.kernel,
        mesh=mesh,
        out_type=jax.ShapeDtypeStruct((B, EMB), jnp.float32),
        scratch_types=[
            pltpu.VMEM((per_w,), jnp.int32),
            pltpu.VMEM((per_w, EMB), jnp.float32),
            pltpu.VMEM((b_per_w, EMB), jnp.float32),
        ],
        compiler_params=pltpu.CompilerParams(use_tc_tiling_on_sc=False),
    )
    def gather_pool(table_hbm, idx_hbm, out_hbm, idx_v, rows_v, emb_v):
        wid = lax.axis_index("s") * _NC + lax.axis_index("c")
        pltpu.sync_copy(idx_hbm.at[wid], idx_v)

        @pl.loop(0, n_chunks)
        def _(c):
            sl = pl.ds(c * CH, CH)
            pltpu.sync_copy(table_hbm.at[idx_v.at[sl]], rows_v.at[sl])

        @pl.loop(0, b_per_w)
        def _(j):
            base = j * CTX
            acc = rows_v[base]
            for c in range(1, CTX):
                acc = acc + rows_v[base + c]
            emb_v[j] = acc

        pltpu.sync_copy(emb_v, out_hbm.at[pl.ds(wid * b_per_w, b_per_w)])

    return gather_pool(emb_table, idx_flat)


def _mm_body(wt_ref, emb_ref, out_ref):
    out_ref[...] = lax.dot_general(
        wt_ref[...], emb_ref[...],
        dimension_numbers=(((0,), (1,)), ((), ())),
        preferred_element_type=jnp.float32,
    )


def _project_tc(embedded, W, b):
    B, EMB = embedded.shape
    V = W.shape[0]
    K = EMB + 1
    wt_aug = jnp.concatenate([W.T, b[None, :]], axis=0)
    emb_aug = jnp.concatenate(
        [embedded, jnp.ones((B, 1), jnp.float32)], axis=1)
    TV = 2048
    grid = pl.cdiv(V, TV)
    out_t = pl.pallas_call(
        _mm_body,
        grid=(grid,),
        in_specs=[
            pl.BlockSpec((K, TV), lambda i: (0, i)),
            pl.BlockSpec((B, K), lambda i: (0, 0)),
        ],
        out_specs=pl.BlockSpec((TV, B), lambda i: (i, 0)),
        out_shape=jax.ShapeDtypeStruct((V, B), jnp.float32),
        compiler_params=pltpu.CompilerParams(
            dimension_semantics=("parallel",),
        ),
    )(wt_aug, emb_aug)
    return out_t.T


def kernel(context_idxs, emb_table, W, b):
    embedded = _embed_pool_sc(context_idxs, emb_table)
    return _project_tc(embedded, W, b)


# speedup vs baseline: 2.9646x; 1.0639x over previous
import functools

import jax
import jax.numpy as jnp
from jax import lax
from jax.experimental import pallas as pl
from jax.experimental.pallas import tpu as pltpu
from jax.experimental.pallas import tpu_sc as plsc

_NC = 2
_NS = 16


def _embed_pool_sc(context_idxs, emb_table):
    B, CTX = context_idxs.shape
    _, EMB = emb_table.shape
    NW = _NC * _NS
    b_per_w = B // NW

    idx_t = context_idxs.astype(jnp.int32).T
    mesh = plsc.VectorSubcoreMesh(core_axis_name="c", subcore_axis_name="s")

    @functools.partial(
        pl.kernel,
        mesh=mesh,
        out_type=jax.ShapeDtypeStruct((B, EMB), jnp.float32),
        scratch_types=[
            pltpu.VMEM((CTX, b_per_w), jnp.int32),
            pltpu.VMEM((CTX * b_per_w, EMB), jnp.float32),
            pltpu.VMEM((b_per_w, EMB), jnp.float32),
            pltpu.SemaphoreType.DMA,
        ],
        compiler_params=pltpu.CompilerParams(use_tc_tiling_on_sc=False),
    )
    def gather_pool(table_hbm, idx_hbm, out_hbm, idx_v, rows_v, emb_v, sem):
        wid = lax.axis_index("s") * _NC + lax.axis_index("c")
        base = wid * b_per_w
        pltpu.sync_copy(idx_hbm.at[:, pl.ds(base, b_per_w)], idx_v)

        @pl.loop(0, CTX)
        def _(c):
            pltpu.make_async_copy(
                table_hbm.at[idx_v.at[c]],
                rows_v.at[pl.ds(c * b_per_w, b_per_w)], sem).start()

        @pl.loop(0, CTX)
        def _(c):
            pltpu.make_async_copy(
                table_hbm.at[idx_v.at[c]],
                rows_v.at[pl.ds(c * b_per_w, b_per_w)], sem).wait()

        @pl.loop(0, b_per_w)
        def _(j):
            acc = rows_v[j]
            for c in range(1, CTX):
                acc = acc + rows_v[c * b_per_w + j]
            emb_v[j] = acc

        pltpu.sync_copy(emb_v, out_hbm.at[pl.ds(base, b_per_w)])

    return gather_pool(emb_table, idx_t)


def _mm_body(wt_ref, emb_ref, out_ref):
    out_ref[...] = lax.dot_general(
        wt_ref[...], emb_ref[...],
        dimension_numbers=(((0,), (1,)), ((), ())),
        preferred_element_type=jnp.float32,
    )


def _project_tc(embedded, W, b):
    B, EMB = embedded.shape
    V = W.shape[0]
    K = EMB + 1
    wt_aug = jnp.concatenate([W.T, b[None, :]], axis=0)
    emb_aug = jnp.concatenate(
        [embedded, jnp.ones((B, 1), jnp.float32)], axis=1)
    TV = 2048
    grid = pl.cdiv(V, TV)
    out_t = pl.pallas_call(
        _mm_body,
        grid=(grid,),
        in_specs=[
            pl.BlockSpec((K, TV), lambda i: (0, i)),
            pl.BlockSpec((B, K), lambda i: (0, 0)),
        ],
        out_specs=pl.BlockSpec((TV, B), lambda i: (i, 0)),
        out_shape=jax.ShapeDtypeStruct((V, B), jnp.float32),
        compiler_params=pltpu.CompilerParams(
            dimension_semantics=("parallel",),
        ),
    )(wt_aug, emb_aug)
    return out_t.T


def kernel(context_idxs, emb_table, W, b):
    embedded = _embed_pool_sc(context_idxs, emb_table)
    return _project_tc(embedded, W, b)
